# baseline (speedup 1.0000x reference)
import functools

import jax
import jax.numpy as jnp
from jax import lax
from jax.experimental import pallas as pl
from jax.experimental.pallas import tpu as pltpu

_N_CB = 8
_K = 1024
_D = 64


def _rvq_kernel(xt_ref, cb_ref, cbs_ref, quant_ref, idx_ref, loss_ref):
    r0 = xt_ref[...]
    r = r0
    loss = jnp.zeros((), dtype=jnp.float32)
    n_rows = r.shape[0]
    iota = lax.broadcasted_iota(jnp.int32, (n_rows, _K), 1).astype(jnp.float32)
    n_ch = 8
    cw = _K // n_ch
    iota_c = lax.broadcasted_iota(jnp.int32, (n_rows, n_ch), 1).astype(jnp.float32)
    iota_w = lax.broadcasted_iota(jnp.int32, (n_rows, cw), 1).astype(jnp.float32)

    cb = cb_ref[...]
    b2 = jnp.sum(cb * cb, axis=2)
    cb_split = cbs_ref[...]

    for i in range(_N_CB):
        prod = jnp.dot(-2.0 * r, cb[i].T, preferred_element_type=jnp.float32)
        d2 = prod + b2[i][None, :]
        chunks = [d2[:, c * cw : (c + 1) * cw] for c in range(n_ch)]
        cmins = jnp.concatenate(
            [jnp.min(ch, axis=1, keepdims=True) for ch in chunks], axis=1)
        m = jnp.min(cmins, axis=1, keepdims=True)
        cidx = jnp.min(jnp.where(cmins == m, iota_c, float(n_ch)), axis=1,
                       keepdims=True)
        ci = cidx.astype(jnp.int32)
        b0 = (ci & 1) != 0
        b1 = (ci & 2) != 0
        b2_ = (ci & 4) != 0
        t0 = [jnp.where(b0, chunks[2 * j + 1], chunks[2 * j]) for j in range(4)]
        u0 = [jnp.where(b1, t0[2 * j + 1], t0[2 * j]) for j in range(2)]
        win = jnp.where(b2_, u0[1], u0[0])
        inpos = jnp.min(jnp.where(win == m, iota_w, float(cw)), axis=1,
                        keepdims=True)
        midx = cidx * float(cw) + inpos
        idx_ref[:, i : i + 1] = midx.astype(jnp.int32)
        onehot = (iota == midx).astype(jnp.bfloat16)
        s = jnp.dot(onehot, cb_split[i], preferred_element_type=jnp.float32)
        q = (s[:, :_D] + s[:, _D : 2 * _D]) + s[:, 2 * _D :]
        r = r - q
        loss = loss + jnp.sum((r - q) ** 2)
    quant_ref[...] = r0 - r
    loss_ref[...] = loss.reshape(1, 1, 1)


@functools.partial(jax.jit, static_argnames=())
def kernel(x, codebooks):
    b, d, t = x.shape
    n_cb, k, dc = codebooks.shape
    n = b * t
    xt = jnp.transpose(x, (0, 2, 1)).reshape(n, d)

    p1 = codebooks.astype(jnp.bfloat16)
    rem = codebooks - p1.astype(jnp.float32)
    p2 = rem.astype(jnp.bfloat16)
    p3 = (rem - p2.astype(jnp.float32)).astype(jnp.bfloat16)
    cb_split = jnp.concatenate([p1, p2, p3], axis=2)

    tile = 1024
    grid = n // tile

    quant, idx, loss_parts = pl.pallas_call(
        _rvq_kernel,
        grid=(grid,),
        in_specs=[
            pl.BlockSpec((tile, d), lambda i: (i, 0)),
            pl.BlockSpec((n_cb, k, dc), lambda i: (0, 0, 0)),
            pl.BlockSpec((n_cb, k, 3 * dc), lambda i: (0, 0, 0)),
        ],
        out_specs=[
            pl.BlockSpec((tile, d), lambda i: (i, 0)),
            pl.BlockSpec((tile, n_cb), lambda i: (i, 0)),
            pl.BlockSpec((1, 1, 1), lambda i: (i, 0, 0)),
        ],
        out_shape=[
            jax.ShapeDtypeStruct((n, d), jnp.float32),
            jax.ShapeDtypeStruct((n, n_cb), jnp.int32),
            jax.ShapeDtypeStruct((grid, 1, 1), jnp.float32),
        ],
        compiler_params=pltpu.CompilerParams(
            dimension_semantics=("parallel",),
        ),
    )(xt, codebooks, cb_split)

    quantized = jnp.transpose(quant.reshape(b, t, d), (0, 2, 1))
    indices = jnp.transpose(idx.reshape(b, t, n_cb), (0, 2, 1))
    commitment_loss = jnp.sum(loss_parts) / jnp.float32(b * t * d)
    return quantized, indices, commitment_loss


# speedup vs baseline: 1.9527x; 1.9527x over previous
import functools

import jax
import jax.numpy as jnp
from jax import lax
from jax.experimental import pallas as pl
from jax.experimental.pallas import tpu as pltpu

_N_CB = 8
_K = 1024
_D = 64


def _rvq_kernel(x_ref, cb_ref, cbst_ref, quant_ref, idx_ref, loss_ref):
    rt0 = x_ref[0]
    rt = rt0
    tt = rt.shape[1]
    loss = jnp.zeros((), dtype=jnp.float32)
    iota = lax.broadcasted_iota(jnp.int32, (_K, tt), 0).astype(jnp.float32)

    cb = cb_ref[...]
    b2 = jnp.sum(cb * cb, axis=2)

    for i in range(_N_CB):
        prod = jnp.dot(cb[i], -2.0 * rt, preferred_element_type=jnp.float32)
        d2 = prod + b2[i][:, None]
        m = jnp.min(d2, axis=0, keepdims=True)
        midx = jnp.min(jnp.where(d2 == m, iota, float(_K)), axis=0,
                       keepdims=True)
        idx_ref[0, i : i + 1, :] = midx.astype(jnp.int32)
        onehot = (iota == midx).astype(jnp.bfloat16)
        s = jnp.dot(cbst_ref[i], onehot, preferred_element_type=jnp.float32)
        q = (s[:_D, :] + s[_D : 2 * _D, :]) + s[2 * _D :, :]
        rt = rt - q
        loss = loss + jnp.sum((rt - q) ** 2)
    quant_ref[0] = rt0 - rt
    loss_ref[...] = loss.reshape(1, 1, 1, 1)


@functools.partial(jax.jit, static_argnames=())
def kernel(x, codebooks):
    b, d, t = x.shape
    n_cb, k, dc = codebooks.shape

    p1 = codebooks.astype(jnp.bfloat16)
    rem = codebooks - p1.astype(jnp.float32)
    p2 = rem.astype(jnp.bfloat16)
    p3 = (rem - p2.astype(jnp.float32)).astype(jnp.bfloat16)
    cb_split_t = jnp.transpose(
        jnp.concatenate([p1, p2, p3], axis=2), (0, 2, 1))

    tile_t = 1024
    tpb = t // tile_t

    quant, idx, loss_parts = pl.pallas_call(
        _rvq_kernel,
        grid=(b, tpb),
        in_specs=[
            pl.BlockSpec((1, d, tile_t), lambda i, j: (i, 0, j)),
            pl.BlockSpec((n_cb, k, dc), lambda i, j: (0, 0, 0)),
            pl.BlockSpec((n_cb, 3 * dc, k), lambda i, j: (0, 0, 0)),
        ],
        out_specs=[
            pl.BlockSpec((1, d, tile_t), lambda i, j: (i, 0, j)),
            pl.BlockSpec((1, n_cb, tile_t), lambda i, j: (i, 0, j)),
            pl.BlockSpec((1, 1, 1, 1), lambda i, j: (i, j, 0, 0)),
        ],
        out_shape=[
            jax.ShapeDtypeStruct((b, d, t), jnp.float32),
            jax.ShapeDtypeStruct((b, n_cb, t), jnp.int32),
            jax.ShapeDtypeStruct((b, tpb, 1, 1), jnp.float32),
        ],
        compiler_params=pltpu.CompilerParams(
            dimension_semantics=("parallel", "parallel"),
        ),
    )(x, codebooks, cb_split_t)

    commitment_loss = jnp.sum(loss_parts) / jnp.float32(b * t * d)
    return quant, idx, commitment_loss
